# Initial kernel scaffold; baseline (speedup 1.0000x reference)
import jax
import jax.numpy as jnp
from jax.experimental import pallas as pl


def kernel(field, target, step, emb_table, g_bn, b_bn, W_ctrl, b_ctrl, g_ctrl, be_ctrl, W1, b1, g1, be1, W2, b2, g2, be2, Wo, bo):
    raise NotImplementedError("write your pallas kernel here")


# speedup vs baseline: 2.2469x; 2.2469x over previous
import functools

import jax
import jax.numpy as jnp
from jax import lax
from jax.experimental import pallas as pl
from jax.experimental.pallas import tpu as pltpu
import jax.experimental.pallas.tpu_sc as plsc

F = 26
D = 16
BATCH = 16384
DIN = F * D
K = 13
EPS = 1e-5
H1 = 512
H2 = 256
FIELD_SIZE = 40000

BM = 1024
NB = BATCH // BM

TOT = BATCH * F
NW = 32
PER_W = TOT // NW
CHUNK = 128
SUB = 8
GROUP = SUB * CHUNK
NG = PER_W // GROUP


def _sc_gather(table, idx):
    idx2 = idx.reshape(TOT // CHUNK, CHUNK)
    mesh = plsc.VectorSubcoreMesh(core_axis_name="c", subcore_axis_name="s")

    @functools.partial(
        pl.kernel,
        out_type=jax.ShapeDtypeStruct((TOT, D), jnp.float32),
        mesh=mesh,
        compiler_params=pltpu.CompilerParams(use_tc_tiling_on_sc=False),
        scratch_types=[
            pltpu.VMEM((PER_W // CHUNK, CHUNK), jnp.int32),
            pltpu.VMEM((GROUP, D), jnp.float32),
            pltpu.SemaphoreType.DMA,
        ],
    )
    def gk(idx_hbm, tab_hbm, out_hbm, idx_v, rows_v, sem):
        wid = lax.axis_index("s") * 2 + lax.axis_index("c")
        row0 = wid * (PER_W // CHUNK)
        base = wid * PER_W
        pltpu.sync_copy(idx_hbm.at[pl.ds(row0, PER_W // CHUNK)], idx_v)

        def body(g, carry):
            cps = []
            for j in range(SUB):
                cp = pltpu.async_copy(
                    tab_hbm.at[idx_v.at[g * SUB + j]],
                    rows_v.at[pl.ds(j * CHUNK, CHUNK)],
                    sem,
                )
                cps.append(cp)
            for cp in cps:
                cp.wait()
            pltpu.sync_copy(rows_v, out_hbm.at[pl.ds(base + g * GROUP, GROUP)])
            return carry

        lax.fori_loop(0, NG, body, 0)

    return gk(idx2, table)


def _stats_body(x_ref, s_ref):
    i = pl.program_id(0)
    x = x_ref[...]

    @pl.when(i == 0)
    def _():
        s_ref[...] = jnp.zeros_like(s_ref)

    s_ref[0:1, :] += jnp.sum(x, axis=0, keepdims=True)
    s_ref[1:2, :] += jnp.sum(x * x, axis=0, keepdims=True)


def _ctrl_body(x_ref, a_ref, w_ref, h_ref, s_ref):
    i = pl.program_id(0)
    x = x_ref[...] * a_ref[...]
    h = jnp.dot(x, w_ref[...], preferred_element_type=jnp.float32)
    h_ref[...] = h

    @pl.when(i == 0)
    def _():
        s_ref[...] = jnp.zeros_like(s_ref)

    s_ref[0:1, :] += jnp.sum(h, axis=0, keepdims=True)
    s_ref[1:2, :] += jnp.sum(h * h, axis=0, keepdims=True)


def _mask_body(x_ref, h_ref, a_ref, c_ref, pq_ref, e_ref, w1_ref, y_ref, s_ref):
    i = pl.program_id(0)
    h = h_ref[...]
    hb = jnp.maximum(h * pq_ref[0:1, :] + pq_ref[1:2, :], 0.0)
    m = jnp.max(hb, axis=1, keepdims=True)
    e = jnp.exp(hb - m)
    w = e / jnp.sum(e, axis=1, keepdims=True)
    iota = lax.broadcasted_iota(jnp.int32, w.shape, 1)
    key = (lax.bitcast_convert_type(w, jnp.int32) & jnp.int32(~31)) | (31 - iota)
    sel = jnp.zeros(w.shape, dtype=jnp.bool_)
    for _ in range(K):
        mx = jnp.max(key, axis=1, keepdims=True)
        chosen = key == mx
        sel = sel | chosen
        key = jnp.where(chosen, jnp.int32(-1), key)
    wsel = jnp.where(sel, w, 0.0)
    maskw = wsel / jnp.sum(wsel, axis=1, keepdims=True)
    maskexp = jnp.dot(maskw, e_ref[...], preferred_element_type=jnp.float32)
    x = (x_ref[...] * a_ref[...] + c_ref[...]) * maskexp
    y = jnp.dot(x, w1_ref[...], preferred_element_type=jnp.float32)
    y_ref[...] = y

    @pl.when(i == 0)
    def _():
        s_ref[...] = jnp.zeros_like(s_ref)

    s_ref[0:1, :] += jnp.sum(y, axis=0, keepdims=True)
    s_ref[1:2, :] += jnp.sum(y * y, axis=0, keepdims=True)


def _mlp_body(x_ref, pq_ref, w_ref, y_ref, s_ref):
    i = pl.program_id(0)
    z = jnp.maximum(x_ref[...] * pq_ref[0:1, :] + pq_ref[1:2, :], 0.0)
    y = jnp.dot(z, w_ref[...], preferred_element_type=jnp.float32)
    y_ref[...] = y

    @pl.when(i == 0)
    def _():
        s_ref[...] = jnp.zeros_like(s_ref)

    s_ref[0:1, :] += jnp.sum(y, axis=0, keepdims=True)
    s_ref[1:2, :] += jnp.sum(y * y, axis=0, keepdims=True)


def _loss_body(x_ref, pq_ref, wo_ref, bo_ref, t_ref, s_ref):
    i = pl.program_id(0)
    z = jnp.maximum(x_ref[...] * pq_ref[0:1, :] + pq_ref[1:2, :], 0.0)
    o = jnp.sum(z * wo_ref[...], axis=1, keepdims=True) + bo_ref[0, 0]
    r = 1.0 / (1.0 + jnp.exp(-o))
    rc = jnp.clip(r, 1e-7, 1.0 - 1e-7)
    t = t_ref[...]
    part = jnp.sum(t * jnp.log(rc) + (1.0 - t) * jnp.log(1.0 - rc))

    @pl.when(i == 0)
    def _():
        s_ref[...] = jnp.zeros_like(s_ref)

    s_ref[...] += part.reshape(1, 1)


def _pq(ssum, ssq, g, be, n):
    mu = ssum / n
    var = ssq / n - mu * mu
    p = g * lax.rsqrt(var + EPS)
    return jnp.stack([p, be - mu * p])


def kernel(field, target, step, emb_table, g_bn, b_bn, W_ctrl, b_ctrl, g_ctrl,
           be_ctrl, W1, b1, g1, be1, W2, b2, g2, be2, Wo, bo):
    offsets = jnp.arange(F, dtype=jnp.int32) * FIELD_SIZE
    idx = (field + offsets[None, :]).reshape(-1)

    raw = _sc_gather(emb_table, idx)
    raw2 = raw.reshape(BATCH, DIN)

    c = jnp.arange(DIN)
    perm = (c % D) * F + c // D
    Wc_p = W_ctrl[perm]
    W1_p = W1[perm]
    expand = (jnp.arange(DIN)[None, :] // D == jnp.arange(F)[:, None]).astype(jnp.float32)

    row_spec = pl.BlockSpec((BM, DIN), lambda i: (i, 0))
    grid = (NB,)

    sums = pl.pallas_call(
        _stats_body,
        grid=grid,
        in_specs=[row_spec],
        out_specs=pl.BlockSpec((2, DIN), lambda i: (0, 0)),
        out_shape=jax.ShapeDtypeStruct((2, DIN), jnp.float32),
    )(raw2)

    n3 = float(BATCH * F)
    m_d = jnp.sum(sums[0].reshape(F, D), axis=0) / n3
    v_d = jnp.sum(sums[1].reshape(F, D), axis=0) / n3 - m_d * m_d
    inv_d = lax.rsqrt(v_d + EPS)
    a_col = jnp.tile(g_bn * inv_d, F)[None, :]
    c_col = jnp.tile(b_bn - g_bn * m_d * inv_d, F)[None, :]

    h, hstats = pl.pallas_call(
        _ctrl_body,
        grid=grid,
        in_specs=[
            row_spec,
            pl.BlockSpec((1, DIN), lambda i: (0, 0)),
            pl.BlockSpec((DIN, F), lambda i: (0, 0)),
        ],
        out_specs=[
            pl.BlockSpec((BM, F), lambda i: (i, 0)),
            pl.BlockSpec((2, F), lambda i: (0, 0)),
        ],
        out_shape=[
            jax.ShapeDtypeStruct((BATCH, F), jnp.float32),
            jax.ShapeDtypeStruct((2, F), jnp.float32),
        ],
    )(raw2, a_col, Wc_p)

    pq_h = _pq(hstats[0], hstats[1], g_ctrl, be_ctrl, float(BATCH))

    y1, s1 = pl.pallas_call(
        _mask_body,
        grid=grid,
        in_specs=[
            row_spec,
            pl.BlockSpec((BM, F), lambda i: (i, 0)),
            pl.BlockSpec((1, DIN), lambda i: (0, 0)),
            pl.BlockSpec((1, DIN), lambda i: (0, 0)),
            pl.BlockSpec((2, F), lambda i: (0, 0)),
            pl.BlockSpec((F, DIN), lambda i: (0, 0)),
            pl.BlockSpec((DIN, H1), lambda i: (0, 0)),
        ],
        out_specs=[
            pl.BlockSpec((BM, H1), lambda i: (i, 0)),
            pl.BlockSpec((2, H1), lambda i: (0, 0)),
        ],
        out_shape=[
            jax.ShapeDtypeStruct((BATCH, H1), jnp.float32),
            jax.ShapeDtypeStruct((2, H1), jnp.float32),
        ],
    )(raw2, h, a_col, c_col, pq_h, expand, W1_p)

    pq1 = _pq(s1[0], s1[1], g1, be1, float(BATCH))

    y2, s2 = pl.pallas_call(
        _mlp_body,
        grid=grid,
        in_specs=[
            pl.BlockSpec((BM, H1), lambda i: (i, 0)),
            pl.BlockSpec((2, H1), lambda i: (0, 0)),
            pl.BlockSpec((H1, H2), lambda i: (0, 0)),
        ],
        out_specs=[
            pl.BlockSpec((BM, H2), lambda i: (i, 0)),
            pl.BlockSpec((2, H2), lambda i: (0, 0)),
        ],
        out_shape=[
            jax.ShapeDtypeStruct((BATCH, H2), jnp.float32),
            jax.ShapeDtypeStruct((2, H2), jnp.float32),
        ],
    )(y1, pq1, W2)

    pq2 = _pq(s2[0], s2[1], g2, be2, float(BATCH))

    acc = pl.pallas_call(
        _loss_body,
        grid=grid,
        in_specs=[
            pl.BlockSpec((BM, H2), lambda i: (i, 0)),
            pl.BlockSpec((2, H2), lambda i: (0, 0)),
            pl.BlockSpec((1, H2), lambda i: (0, 0)),
            pl.BlockSpec((1, 1), lambda i: (0, 0)),
            pl.BlockSpec((BM, 1), lambda i: (i, 0)),
        ],
        out_specs=pl.BlockSpec((1, 1), lambda i: (0, 0)),
        out_shape=jax.ShapeDtypeStruct((1, 1), jnp.float32),
    )(y2, pq2, Wo.T, bo.reshape(1, 1), target.reshape(BATCH, 1))

    return -acc[0, 0] / BATCH
